# Initial kernel scaffold; baseline (speedup 1.0000x reference)
import jax
import jax.numpy as jnp
from jax.experimental import pallas as pl


def kernel(unknown, known, unknow_feats, known_feats, W0, gamma0, beta0, W1, gamma1, beta1, W2, gamma2, beta2):
    raise NotImplementedError("write your pallas kernel here")


# speedup vs baseline: 1133.4290x; 1133.4290x over previous
import functools

import jax
import jax.numpy as jnp
from jax.experimental import pallas as pl


def _nn_interp_kernel(ut_ref, kn_ref, kf_ref, out_ref, *, M):
    ut = ut_ref[0]
    kn = kn_ref[0]
    u2 = jnp.sum(ut * ut, axis=0, keepdims=True)
    k2 = jnp.sum(kn * kn, axis=1, keepdims=True)
    d2 = u2 + k2 - 2.0 * jnp.dot(kn.astype(jnp.bfloat16), ut.astype(jnp.bfloat16), preferred_element_type=jnp.float32)
    miota = jax.lax.broadcasted_iota(jnp.int32, d2.shape, 0)
    cur = d2
    dists, idxs = [], []
    for _ in range(3):
        mn = jnp.min(cur, axis=0, keepdims=True)
        ik = jnp.min(jnp.where(cur == mn, miota, M), axis=0, keepdims=True)
        cur = jnp.where(miota == ik, jnp.float32(jnp.inf), cur)
        dists.append(jnp.maximum(mn, 0.0))
        idxs.append(ik)
    r = [1.0 / (d + 1e-8) for d in dists]
    norm = r[0] + r[1] + r[2]
    S = (
        jnp.where(miota == idxs[0], r[0] / norm, 0.0)
        + jnp.where(miota == idxs[1], r[1] / norm, 0.0)
        + jnp.where(miota == idxs[2], r[2] / norm, 0.0)
    )
    out_ref[0] = jnp.dot(kf_ref[0], S, preferred_element_type=jnp.float32)


def _accum_stats(st_ref, y):
    @pl.when((pl.program_id(0) == 0) & (pl.program_id(1) == 0))
    def _():
        st_ref[...] = jnp.zeros_like(st_ref)

    st_ref[:, 0:1] += jnp.sum(y, axis=1, keepdims=True)
    st_ref[:, 1:2] += jnp.sum(y * y, axis=1, keepdims=True)


def _layer0_kernel(interp_ref, uf_ref, w_ref, y_ref, st_ref, *, C2):
    y = jnp.dot(w_ref[:, :C2], interp_ref[0], preferred_element_type=jnp.float32)
    y += jnp.dot(w_ref[:, C2:], uf_ref[0], preferred_element_type=jnp.float32)
    y_ref[0] = y
    _accum_stats(st_ref, y)


def _bn_scale_shift(st, g, b, count):
    mean = st[:, 0:1] * (1.0 / count)
    var = st[:, 1:2] * (1.0 / count) - mean * mean
    scale = g * jax.lax.rsqrt(var + 1e-5)
    return scale, b - mean * scale


def _bn_mm_kernel(x_ref, st0_ref, g_ref, b_ref, w_ref, y_ref, st_ref, *, count):
    scale, shift = _bn_scale_shift(st0_ref[...], g_ref[...], b_ref[...], count)
    z = jnp.maximum(x_ref[0] * scale + shift, 0.0)
    y = jnp.dot(w_ref[...], z, preferred_element_type=jnp.float32)
    y_ref[0] = y
    _accum_stats(st_ref, y)


def _bn_relu_kernel(x_ref, st0_ref, g_ref, b_ref, y_ref, *, count):
    scale, shift = _bn_scale_shift(st0_ref[...], g_ref[...], b_ref[...], count)
    y_ref[0] = jnp.maximum(x_ref[0] * scale + shift, 0.0)


def kernel(unknown, known, unknow_feats, known_feats,
           W0, gamma0, beta0, W1, gamma1, beta1, W2, gamma2, beta2):
    B, N, _ = unknown.shape
    M = known.shape[1]
    C1 = unknow_feats.shape[1]
    C2 = known_feats.shape[1]
    f32 = jnp.float32
    ut = jnp.transpose(unknown, (0, 2, 1))

    TN = min(512, N)
    grid = (B, N // TN)
    interp = pl.pallas_call(
        functools.partial(_nn_interp_kernel, M=M),
        grid=grid,
        in_specs=[
            pl.BlockSpec((1, 3, TN), lambda b, n: (b, 0, n)),
            pl.BlockSpec((1, M, 3), lambda b, n: (b, 0, 0)),
            pl.BlockSpec((1, C2, M), lambda b, n: (b, 0, 0)),
        ],
        out_specs=pl.BlockSpec((1, C2, TN), lambda b, n: (b, 0, n)),
        out_shape=jax.ShapeDtypeStruct((B, C2, N), f32),
    )(ut, known, known_feats)

    count = B * N
    O0, O1, O2 = W0.shape[0], W1.shape[0], W2.shape[0]

    def cm_spec(C):
        return pl.BlockSpec((1, C, TN), lambda b, n: (b, 0, n))

    def full2d(a):
        return pl.BlockSpec(a.shape, lambda b, n: (0, 0))

    y0, st0 = pl.pallas_call(
        functools.partial(_layer0_kernel, C2=C2),
        grid=grid,
        in_specs=[cm_spec(C2), cm_spec(C1), full2d(W0)],
        out_specs=[cm_spec(O0), pl.BlockSpec((O0, 2), lambda b, n: (0, 0))],
        out_shape=[jax.ShapeDtypeStruct((B, O0, N), f32),
                   jax.ShapeDtypeStruct((O0, 2), f32)],
    )(interp, unknow_feats, W0)

    def bn_layer(x, st, g, b, W, Oin, Oout):
        return pl.pallas_call(
            functools.partial(_bn_mm_kernel, count=count),
            grid=grid,
            in_specs=[cm_spec(Oin), pl.BlockSpec((Oin, 2), lambda b, n: (0, 0)),
                      pl.BlockSpec((Oin, 1), lambda b, n: (0, 0)),
                      pl.BlockSpec((Oin, 1), lambda b, n: (0, 0)), full2d(W)],
            out_specs=[cm_spec(Oout), pl.BlockSpec((Oout, 2), lambda b, n: (0, 0))],
            out_shape=[jax.ShapeDtypeStruct((B, Oout, N), f32),
                       jax.ShapeDtypeStruct((Oout, 2), f32)],
        )(x, st, g.reshape(Oin, 1), b.reshape(Oin, 1), W)

    y1, st1 = bn_layer(y0, st0, gamma0, beta0, W1, O0, O1)
    y2, st2 = bn_layer(y1, st1, gamma1, beta1, W2, O1, O2)

    out = pl.pallas_call(
        functools.partial(_bn_relu_kernel, count=count),
        grid=grid,
        in_specs=[cm_spec(O2), pl.BlockSpec((O2, 2), lambda b, n: (0, 0)),
                  pl.BlockSpec((O2, 1), lambda b, n: (0, 0)),
                  pl.BlockSpec((O2, 1), lambda b, n: (0, 0))],
        out_specs=cm_spec(O2),
        out_shape=jax.ShapeDtypeStruct((B, O2, N), f32),
    )(y2, st2, gamma2.reshape(O2, 1), beta2.reshape(O2, 1))
    return out


# speedup vs baseline: 1210.9027x; 1.0684x over previous
import functools

import jax
import jax.numpy as jnp
from jax.experimental import pallas as pl


def _nn_interp_kernel(ut_ref, kn_ref, kf_ref, out_ref, *, M):
    ut = ut_ref[0]
    kn = kn_ref[0]
    u2 = jnp.sum(ut * ut, axis=0, keepdims=True)
    k2 = jnp.sum(kn * kn, axis=1, keepdims=True)
    d2 = u2 + k2 - 2.0 * jnp.dot(kn.astype(jnp.bfloat16), ut.astype(jnp.bfloat16), preferred_element_type=jnp.float32)
    miota = jax.lax.broadcasted_iota(jnp.int32, d2.shape, 0)
    cur = d2
    dists, idxs = [], []
    for _ in range(3):
        mn = jnp.min(cur, axis=0, keepdims=True)
        ik = jnp.min(jnp.where(cur == mn, miota, M), axis=0, keepdims=True)
        cur = jnp.where(miota == ik, jnp.float32(jnp.inf), cur)
        dists.append(jnp.maximum(mn, 0.0))
        idxs.append(ik)
    r = [1.0 / (d + 1e-8) for d in dists]
    norm = r[0] + r[1] + r[2]
    S = (
        jnp.where(miota == idxs[0], r[0] / norm, 0.0)
        + jnp.where(miota == idxs[1], r[1] / norm, 0.0)
        + jnp.where(miota == idxs[2], r[2] / norm, 0.0)
    )
    out_ref[0] = jnp.dot(kf_ref[0], S,
                         preferred_element_type=jnp.float32).astype(out_ref.dtype)


def _accum_stats(st_ref, y):
    @pl.when((pl.program_id(0) == 0) & (pl.program_id(1) == 0))
    def _():
        st_ref[...] = jnp.zeros_like(st_ref)

    st_ref[:, 0:1] += jnp.sum(y, axis=1, keepdims=True)
    st_ref[:, 1:2] += jnp.sum(y * y, axis=1, keepdims=True)


def _layer0_kernel(interp_ref, uf_ref, w_ref, y_ref, st_ref, *, C2):
    y = jnp.dot(w_ref[:, :C2], interp_ref[0], preferred_element_type=jnp.float32)
    y += jnp.dot(w_ref[:, C2:], uf_ref[0].astype(jnp.bfloat16),
                 preferred_element_type=jnp.float32)
    y_ref[0] = y.astype(y_ref.dtype)
    _accum_stats(st_ref, y)


def _bn_scale_shift(st, g, b, count):
    mean = st[:, 0:1] * (1.0 / count)
    var = st[:, 1:2] * (1.0 / count) - mean * mean
    scale = g * jax.lax.rsqrt(var + 1e-5)
    return scale, b - mean * scale


def _bn_mm_kernel(x_ref, st0_ref, g_ref, b_ref, w_ref, y_ref, st_ref, *, count):
    scale, shift = _bn_scale_shift(st0_ref[...], g_ref[...], b_ref[...], count)
    z = jnp.maximum(x_ref[0].astype(jnp.float32) * scale + shift, 0.0)
    y = jnp.dot(w_ref[...], z.astype(jnp.bfloat16),
                preferred_element_type=jnp.float32)
    y_ref[0] = y.astype(y_ref.dtype)
    _accum_stats(st_ref, y)


def _bn_relu_kernel(x_ref, st0_ref, g_ref, b_ref, y_ref, *, count):
    scale, shift = _bn_scale_shift(st0_ref[...], g_ref[...], b_ref[...], count)
    y_ref[0] = jnp.maximum(x_ref[0].astype(jnp.float32) * scale + shift, 0.0)


def kernel(unknown, known, unknow_feats, known_feats,
           W0, gamma0, beta0, W1, gamma1, beta1, W2, gamma2, beta2):
    B, N, _ = unknown.shape
    M = known.shape[1]
    C1 = unknow_feats.shape[1]
    C2 = known_feats.shape[1]
    f32 = jnp.float32
    ut = jnp.transpose(unknown, (0, 2, 1))

    TN = min(512, N)
    grid = (B, N // TN)
    interp = pl.pallas_call(
        functools.partial(_nn_interp_kernel, M=M),
        grid=grid,
        in_specs=[
            pl.BlockSpec((1, 3, TN), lambda b, n: (b, 0, n)),
            pl.BlockSpec((1, M, 3), lambda b, n: (b, 0, 0)),
            pl.BlockSpec((1, C2, M), lambda b, n: (b, 0, 0)),
        ],
        out_specs=pl.BlockSpec((1, C2, TN), lambda b, n: (b, 0, n)),
        out_shape=jax.ShapeDtypeStruct((B, C2, N), jnp.bfloat16),
    )(ut, known, known_feats)

    count = B * N
    O0, O1, O2 = W0.shape[0], W1.shape[0], W2.shape[0]

    def cm_spec(C):
        return pl.BlockSpec((1, C, TN), lambda b, n: (b, 0, n))

    def full2d(a):
        return pl.BlockSpec(a.shape, lambda b, n: (0, 0))

    y0, st0 = pl.pallas_call(
        functools.partial(_layer0_kernel, C2=C2),
        grid=grid,
        in_specs=[cm_spec(C2), cm_spec(C1), full2d(W0)],
        out_specs=[cm_spec(O0), pl.BlockSpec((O0, 2), lambda b, n: (0, 0))],
        out_shape=[jax.ShapeDtypeStruct((B, O0, N), jnp.bfloat16),
                   jax.ShapeDtypeStruct((O0, 2), f32)],
    )(interp, unknow_feats, W0.astype(jnp.bfloat16))

    def bn_layer(x, st, g, b, W, Oin, Oout):
        return pl.pallas_call(
            functools.partial(_bn_mm_kernel, count=count),
            grid=grid,
            in_specs=[cm_spec(Oin), pl.BlockSpec((Oin, 2), lambda b, n: (0, 0)),
                      pl.BlockSpec((Oin, 1), lambda b, n: (0, 0)),
                      pl.BlockSpec((Oin, 1), lambda b, n: (0, 0)), full2d(W)],
            out_specs=[cm_spec(Oout), pl.BlockSpec((Oout, 2), lambda b, n: (0, 0))],
            out_shape=[jax.ShapeDtypeStruct((B, Oout, N), jnp.bfloat16),
                       jax.ShapeDtypeStruct((Oout, 2), f32)],
        )(x, st, g.reshape(Oin, 1), b.reshape(Oin, 1), W.astype(jnp.bfloat16))

    y1, st1 = bn_layer(y0, st0, gamma0, beta0, W1, O0, O1)
    y2, st2 = bn_layer(y1, st1, gamma1, beta1, W2, O1, O2)

    out = pl.pallas_call(
        functools.partial(_bn_relu_kernel, count=count),
        grid=grid,
        in_specs=[cm_spec(O2), pl.BlockSpec((O2, 2), lambda b, n: (0, 0)),
                  pl.BlockSpec((O2, 1), lambda b, n: (0, 0)),
                  pl.BlockSpec((O2, 1), lambda b, n: (0, 0))],
        out_specs=cm_spec(O2),
        out_shape=jax.ShapeDtypeStruct((B, O2, N), f32),
    )(y2, st2, gamma2.reshape(O2, 1), beta2.reshape(O2, 1))
    return out


# speedup vs baseline: 1576.9646x; 1.3023x over previous
import functools

import jax
import jax.numpy as jnp
from jax.experimental import pallas as pl


def _nn_interp_kernel(ut_ref, kn_ref, kf_ref, out_ref, *, M):
    ut = ut_ref[0]
    kn = kn_ref[0]
    u2 = jnp.sum(ut * ut, axis=0, keepdims=True)
    k2 = jnp.sum(kn * kn, axis=1, keepdims=True)
    d2 = u2 + k2 - 2.0 * jnp.dot(kn.astype(jnp.bfloat16), ut.astype(jnp.bfloat16), preferred_element_type=jnp.float32)
    miota = jax.lax.broadcasted_iota(jnp.int32, d2.shape, 0)
    cur = d2
    dists, idxs = [], []
    for _ in range(3):
        mn = jnp.min(cur, axis=0, keepdims=True)
        ik = jnp.min(jnp.where(cur == mn, miota, M), axis=0, keepdims=True)
        cur = jnp.where(miota == ik, jnp.float32(jnp.inf), cur)
        dists.append(jnp.maximum(mn, 0.0))
        idxs.append(ik)
    r = [1.0 / (d + 1e-8) for d in dists]
    norm = r[0] + r[1] + r[2]
    S = (
        jnp.where(miota == idxs[0], r[0] / norm, 0.0)
        + jnp.where(miota == idxs[1], r[1] / norm, 0.0)
        + jnp.where(miota == idxs[2], r[2] / norm, 0.0)
    )
    out_ref[0] = jnp.dot(kf_ref[0], S,
                         preferred_element_type=jnp.float32).astype(out_ref.dtype)


def _accum_stats(st_ref, y):
    @pl.when((pl.program_id(0) == 0) & (pl.program_id(1) == 0))
    def _():
        st_ref[...] = jnp.zeros_like(st_ref)

    st_ref[:, 0:1] += jnp.sum(y, axis=1, keepdims=True)
    st_ref[:, 1:2] += jnp.sum(y * y, axis=1, keepdims=True)


def _layer0_kernel(interp_ref, uf_ref, w_ref, y_ref, st_ref, *, C2):
    y = jnp.dot(w_ref[:, :C2], interp_ref[0], preferred_element_type=jnp.float32)
    y += jnp.dot(w_ref[:, C2:], uf_ref[0].astype(jnp.bfloat16),
                 preferred_element_type=jnp.float32)
    y_ref[0] = y.astype(y_ref.dtype)
    _accum_stats(st_ref, y)


def _bn_scale_shift(st, g, b, count):
    mean = st[:, 0:1] * (1.0 / count)
    var = st[:, 1:2] * (1.0 / count) - mean * mean
    scale = g * jax.lax.rsqrt(var + 1e-5)
    return scale, b - mean * scale


def _bn_mm_kernel(x_ref, st0_ref, g_ref, b_ref, w_ref, y_ref, st_ref, *, count):
    scale, shift = _bn_scale_shift(st0_ref[...], g_ref[...], b_ref[...], count)
    z = jnp.maximum(x_ref[0].astype(jnp.float32) * scale + shift, 0.0)
    y = jnp.dot(w_ref[...], z.astype(jnp.bfloat16),
                preferred_element_type=jnp.float32)
    y_ref[0] = y.astype(y_ref.dtype)
    _accum_stats(st_ref, y)


def _bn_relu_kernel(x_ref, st0_ref, g_ref, b_ref, y_ref, *, count):
    scale, shift = _bn_scale_shift(st0_ref[...], g_ref[...], b_ref[...], count)
    y_ref[0] = jnp.maximum(x_ref[0].astype(jnp.float32) * scale + shift, 0.0)


def kernel(unknown, known, unknow_feats, known_feats,
           W0, gamma0, beta0, W1, gamma1, beta1, W2, gamma2, beta2):
    B, N, _ = unknown.shape
    M = known.shape[1]
    C1 = unknow_feats.shape[1]
    C2 = known_feats.shape[1]
    f32 = jnp.float32
    ut = jnp.transpose(unknown, (0, 2, 1))

    TN = min(1024, N)
    grid = (B, N // TN)
    interp = pl.pallas_call(
        functools.partial(_nn_interp_kernel, M=M),
        grid=grid,
        in_specs=[
            pl.BlockSpec((1, 3, TN), lambda b, n: (b, 0, n)),
            pl.BlockSpec((1, M, 3), lambda b, n: (b, 0, 0)),
            pl.BlockSpec((1, C2, M), lambda b, n: (b, 0, 0)),
        ],
        out_specs=pl.BlockSpec((1, C2, TN), lambda b, n: (b, 0, n)),
        out_shape=jax.ShapeDtypeStruct((B, C2, N), jnp.bfloat16),
    )(ut, known, known_feats)

    count = B * N
    O0, O1, O2 = W0.shape[0], W1.shape[0], W2.shape[0]

    def cm_spec(C):
        return pl.BlockSpec((1, C, TN), lambda b, n: (b, 0, n))

    def full2d(a):
        return pl.BlockSpec(a.shape, lambda b, n: (0, 0))

    y0, st0 = pl.pallas_call(
        functools.partial(_layer0_kernel, C2=C2),
        grid=grid,
        in_specs=[cm_spec(C2), cm_spec(C1), full2d(W0)],
        out_specs=[cm_spec(O0), pl.BlockSpec((O0, 2), lambda b, n: (0, 0))],
        out_shape=[jax.ShapeDtypeStruct((B, O0, N), jnp.bfloat16),
                   jax.ShapeDtypeStruct((O0, 2), f32)],
    )(interp, unknow_feats, W0.astype(jnp.bfloat16))

    def bn_layer(x, st, g, b, W, Oin, Oout):
        return pl.pallas_call(
            functools.partial(_bn_mm_kernel, count=count),
            grid=grid,
            in_specs=[cm_spec(Oin), pl.BlockSpec((Oin, 2), lambda b, n: (0, 0)),
                      pl.BlockSpec((Oin, 1), lambda b, n: (0, 0)),
                      pl.BlockSpec((Oin, 1), lambda b, n: (0, 0)), full2d(W)],
            out_specs=[cm_spec(Oout), pl.BlockSpec((Oout, 2), lambda b, n: (0, 0))],
            out_shape=[jax.ShapeDtypeStruct((B, Oout, N), jnp.bfloat16),
                       jax.ShapeDtypeStruct((Oout, 2), f32)],
        )(x, st, g.reshape(Oin, 1), b.reshape(Oin, 1), W.astype(jnp.bfloat16))

    y1, st1 = bn_layer(y0, st0, gamma0, beta0, W1, O0, O1)
    y2, st2 = bn_layer(y1, st1, gamma1, beta1, W2, O1, O2)

    out = pl.pallas_call(
        functools.partial(_bn_relu_kernel, count=count),
        grid=grid,
        in_specs=[cm_spec(O2), pl.BlockSpec((O2, 2), lambda b, n: (0, 0)),
                  pl.BlockSpec((O2, 1), lambda b, n: (0, 0)),
                  pl.BlockSpec((O2, 1), lambda b, n: (0, 0))],
        out_specs=cm_spec(O2),
        out_shape=jax.ShapeDtypeStruct((B, O2, N), f32),
    )(y2, st2, gamma2.reshape(O2, 1), beta2.reshape(O2, 1))
    return out


# speedup vs baseline: 1821.0208x; 1.1548x over previous
import functools

import jax
import jax.numpy as jnp
from jax.experimental import pallas as pl


def _nn_interp_kernel(ut_ref, kn_ref, kf_ref, out_ref, *, M):
    ut = ut_ref[0]
    kn = kn_ref[0]
    u2 = jnp.sum(ut * ut, axis=0, keepdims=True)
    k2 = jnp.sum(kn * kn, axis=1, keepdims=True)
    d2 = u2 + k2 - 2.0 * jnp.dot(kn.astype(jnp.bfloat16), ut.astype(jnp.bfloat16), preferred_element_type=jnp.float32)
    miota = jax.lax.broadcasted_iota(jnp.int32, d2.shape, 0)
    cur = d2
    dists, idxs = [], []
    for _ in range(3):
        mn = jnp.min(cur, axis=0, keepdims=True)
        ik = jnp.min(jnp.where(cur == mn, miota, M), axis=0, keepdims=True)
        cur = jnp.where(miota == ik, jnp.float32(jnp.inf), cur)
        dists.append(jnp.maximum(mn, 0.0))
        idxs.append(ik)
    r = [1.0 / (d + 1e-8) for d in dists]
    norm = r[0] + r[1] + r[2]
    S = (
        jnp.where(miota == idxs[0], r[0] / norm, 0.0)
        + jnp.where(miota == idxs[1], r[1] / norm, 0.0)
        + jnp.where(miota == idxs[2], r[2] / norm, 0.0)
    )
    out_ref[0] = jnp.dot(kf_ref[0], S,
                         preferred_element_type=jnp.float32).astype(out_ref.dtype)


def _accum_stats(st_ref, y):
    @pl.when((pl.program_id(0) == 0) & (pl.program_id(1) == 0))
    def _():
        st_ref[...] = jnp.zeros_like(st_ref)

    st_ref[:, 0:1] += jnp.sum(y, axis=1, keepdims=True)
    st_ref[:, 1:2] += jnp.sum(y * y, axis=1, keepdims=True)


def _layer0_kernel(interp_ref, uf_ref, w_ref, y_ref, st_ref, *, C2):
    y = jnp.dot(w_ref[:, :C2], interp_ref[0], preferred_element_type=jnp.float32)
    y += jnp.dot(w_ref[:, C2:], uf_ref[0].astype(jnp.bfloat16),
                 preferred_element_type=jnp.float32)
    y_ref[0] = y.astype(y_ref.dtype)
    _accum_stats(st_ref, y)


def _bn_scale_shift(st, g, b, count):
    mean = st[:, 0:1] * (1.0 / count)
    var = st[:, 1:2] * (1.0 / count) - mean * mean
    scale = g * jax.lax.rsqrt(var + 1e-5)
    return scale, b - mean * scale


def _bn_mm_kernel(x_ref, st0_ref, g_ref, b_ref, w_ref, y_ref, st_ref, *, count):
    scale, shift = _bn_scale_shift(st0_ref[...], g_ref[...], b_ref[...], count)
    z = jnp.maximum(x_ref[0].astype(jnp.float32) * scale + shift, 0.0)
    y = jnp.dot(w_ref[...], z.astype(jnp.bfloat16),
                preferred_element_type=jnp.float32)
    y_ref[0] = y.astype(y_ref.dtype)
    _accum_stats(st_ref, y)


def _bn_relu_kernel(x_ref, st0_ref, g_ref, b_ref, y_ref, *, count):
    scale, shift = _bn_scale_shift(st0_ref[...], g_ref[...], b_ref[...], count)
    y_ref[0] = jnp.maximum(x_ref[0].astype(jnp.float32) * scale + shift, 0.0)


def kernel(unknown, known, unknow_feats, known_feats,
           W0, gamma0, beta0, W1, gamma1, beta1, W2, gamma2, beta2):
    B, N, _ = unknown.shape
    M = known.shape[1]
    C1 = unknow_feats.shape[1]
    C2 = known_feats.shape[1]
    f32 = jnp.float32
    ut = jnp.transpose(unknown, (0, 2, 1))

    TN = min(2048, N)
    grid = (B, N // TN)
    interp = pl.pallas_call(
        functools.partial(_nn_interp_kernel, M=M),
        grid=grid,
        in_specs=[
            pl.BlockSpec((1, 3, TN), lambda b, n: (b, 0, n)),
            pl.BlockSpec((1, M, 3), lambda b, n: (b, 0, 0)),
            pl.BlockSpec((1, C2, M), lambda b, n: (b, 0, 0)),
        ],
        out_specs=pl.BlockSpec((1, C2, TN), lambda b, n: (b, 0, n)),
        out_shape=jax.ShapeDtypeStruct((B, C2, N), jnp.bfloat16),
    )(ut, known, known_feats)

    count = B * N
    O0, O1, O2 = W0.shape[0], W1.shape[0], W2.shape[0]

    def cm_spec(C):
        return pl.BlockSpec((1, C, TN), lambda b, n: (b, 0, n))

    def full2d(a):
        return pl.BlockSpec(a.shape, lambda b, n: (0, 0))

    y0, st0 = pl.pallas_call(
        functools.partial(_layer0_kernel, C2=C2),
        grid=grid,
        in_specs=[cm_spec(C2), cm_spec(C1), full2d(W0)],
        out_specs=[cm_spec(O0), pl.BlockSpec((O0, 2), lambda b, n: (0, 0))],
        out_shape=[jax.ShapeDtypeStruct((B, O0, N), jnp.bfloat16),
                   jax.ShapeDtypeStruct((O0, 2), f32)],
    )(interp, unknow_feats, W0.astype(jnp.bfloat16))

    def bn_layer(x, st, g, b, W, Oin, Oout):
        return pl.pallas_call(
            functools.partial(_bn_mm_kernel, count=count),
            grid=grid,
            in_specs=[cm_spec(Oin), pl.BlockSpec((Oin, 2), lambda b, n: (0, 0)),
                      pl.BlockSpec((Oin, 1), lambda b, n: (0, 0)),
                      pl.BlockSpec((Oin, 1), lambda b, n: (0, 0)), full2d(W)],
            out_specs=[cm_spec(Oout), pl.BlockSpec((Oout, 2), lambda b, n: (0, 0))],
            out_shape=[jax.ShapeDtypeStruct((B, Oout, N), jnp.bfloat16),
                       jax.ShapeDtypeStruct((Oout, 2), f32)],
        )(x, st, g.reshape(Oin, 1), b.reshape(Oin, 1), W.astype(jnp.bfloat16))

    y1, st1 = bn_layer(y0, st0, gamma0, beta0, W1, O0, O1)
    y2, st2 = bn_layer(y1, st1, gamma1, beta1, W2, O1, O2)

    out = pl.pallas_call(
        functools.partial(_bn_relu_kernel, count=count),
        grid=grid,
        in_specs=[cm_spec(O2), pl.BlockSpec((O2, 2), lambda b, n: (0, 0)),
                  pl.BlockSpec((O2, 1), lambda b, n: (0, 0)),
                  pl.BlockSpec((O2, 1), lambda b, n: (0, 0))],
        out_specs=cm_spec(O2),
        out_shape=jax.ShapeDtypeStruct((B, O2, N), f32),
    )(y2, st2, gamma2.reshape(O2, 1), beta2.reshape(O2, 1))
    return out


# speedup vs baseline: 1971.4019x; 1.0826x over previous
import functools

import jax
import jax.numpy as jnp
from jax.experimental import pallas as pl


def _nn_interp_kernel(ut_ref, kn_ref, kf_ref, out_ref, *, M):
    ut = ut_ref[0]
    kn = kn_ref[0]
    u2 = jnp.sum(ut * ut, axis=0, keepdims=True)
    k2 = jnp.sum(kn * kn, axis=1, keepdims=True)
    d2 = u2 + k2 - 2.0 * jnp.dot(kn.astype(jnp.bfloat16), ut.astype(jnp.bfloat16), preferred_element_type=jnp.float32)
    miota = jax.lax.broadcasted_iota(jnp.int32, d2.shape, 0)
    cur = d2
    dists, idxs = [], []
    for _ in range(3):
        mn = jnp.min(cur, axis=0, keepdims=True)
        ik = jnp.min(jnp.where(cur == mn, miota, M), axis=0, keepdims=True)
        cur = jnp.where(miota == ik, jnp.float32(jnp.inf), cur)
        dists.append(jnp.maximum(mn, 0.0))
        idxs.append(ik)
    r = [1.0 / (d + 1e-8) for d in dists]
    norm = r[0] + r[1] + r[2]
    S = (
        jnp.where(miota == idxs[0], r[0] / norm, 0.0)
        + jnp.where(miota == idxs[1], r[1] / norm, 0.0)
        + jnp.where(miota == idxs[2], r[2] / norm, 0.0)
    )
    out_ref[0] = jnp.dot(kf_ref[0], S,
                         preferred_element_type=jnp.float32).astype(out_ref.dtype)


def _accum_stats(st_ref, y):
    @pl.when((pl.program_id(0) == 0) & (pl.program_id(1) == 0))
    def _():
        st_ref[...] = jnp.zeros_like(st_ref)

    st_ref[:, 0:1] += jnp.sum(y, axis=1, keepdims=True)
    st_ref[:, 1:2] += jnp.sum(y * y, axis=1, keepdims=True)


def _layer0_kernel(interp_ref, uf_ref, w_ref, y_ref, st_ref, *, C2):
    y = jnp.dot(w_ref[:, :C2], interp_ref[0], preferred_element_type=jnp.float32)
    y += jnp.dot(w_ref[:, C2:], uf_ref[0].astype(jnp.bfloat16),
                 preferred_element_type=jnp.float32)
    y_ref[0] = y.astype(y_ref.dtype)
    _accum_stats(st_ref, y)


def _bn_scale_shift(st, g, b, count):
    mean = st[:, 0:1] * (1.0 / count)
    var = st[:, 1:2] * (1.0 / count) - mean * mean
    scale = g * jax.lax.rsqrt(var + 1e-5)
    return scale, b - mean * scale


def _bn_mm_kernel(x_ref, st0_ref, g_ref, b_ref, w_ref, y_ref, st_ref, *, count):
    scale, shift = _bn_scale_shift(st0_ref[...], g_ref[...], b_ref[...], count)
    z = jnp.maximum(x_ref[0].astype(jnp.float32) * scale + shift, 0.0)
    y = jnp.dot(w_ref[...], z.astype(jnp.bfloat16),
                preferred_element_type=jnp.float32)
    y_ref[0] = y.astype(y_ref.dtype)
    _accum_stats(st_ref, y)


def _bn_relu_kernel(x_ref, st0_ref, g_ref, b_ref, y_ref, *, count):
    scale, shift = _bn_scale_shift(st0_ref[...], g_ref[...], b_ref[...], count)
    y_ref[0] = jnp.maximum(x_ref[0].astype(jnp.float32) * scale + shift, 0.0)


def kernel(unknown, known, unknow_feats, known_feats,
           W0, gamma0, beta0, W1, gamma1, beta1, W2, gamma2, beta2):
    B, N, _ = unknown.shape
    M = known.shape[1]
    C1 = unknow_feats.shape[1]
    C2 = known_feats.shape[1]
    f32 = jnp.float32
    ut = jnp.transpose(unknown, (0, 2, 1))

    TN = min(4096, N)
    grid = (B, N // TN)
    interp = pl.pallas_call(
        functools.partial(_nn_interp_kernel, M=M),
        grid=grid,
        in_specs=[
            pl.BlockSpec((1, 3, TN), lambda b, n: (b, 0, n)),
            pl.BlockSpec((1, M, 3), lambda b, n: (b, 0, 0)),
            pl.BlockSpec((1, C2, M), lambda b, n: (b, 0, 0)),
        ],
        out_specs=pl.BlockSpec((1, C2, TN), lambda b, n: (b, 0, n)),
        out_shape=jax.ShapeDtypeStruct((B, C2, N), jnp.bfloat16),
    )(ut, known, known_feats)

    count = B * N
    O0, O1, O2 = W0.shape[0], W1.shape[0], W2.shape[0]

    def cm_spec(C):
        return pl.BlockSpec((1, C, TN), lambda b, n: (b, 0, n))

    def full2d(a):
        return pl.BlockSpec(a.shape, lambda b, n: (0, 0))

    y0, st0 = pl.pallas_call(
        functools.partial(_layer0_kernel, C2=C2),
        grid=grid,
        in_specs=[cm_spec(C2), cm_spec(C1), full2d(W0)],
        out_specs=[cm_spec(O0), pl.BlockSpec((O0, 2), lambda b, n: (0, 0))],
        out_shape=[jax.ShapeDtypeStruct((B, O0, N), jnp.bfloat16),
                   jax.ShapeDtypeStruct((O0, 2), f32)],
    )(interp, unknow_feats, W0.astype(jnp.bfloat16))

    def bn_layer(x, st, g, b, W, Oin, Oout):
        return pl.pallas_call(
            functools.partial(_bn_mm_kernel, count=count),
            grid=grid,
            in_specs=[cm_spec(Oin), pl.BlockSpec((Oin, 2), lambda b, n: (0, 0)),
                      pl.BlockSpec((Oin, 1), lambda b, n: (0, 0)),
                      pl.BlockSpec((Oin, 1), lambda b, n: (0, 0)), full2d(W)],
            out_specs=[cm_spec(Oout), pl.BlockSpec((Oout, 2), lambda b, n: (0, 0))],
            out_shape=[jax.ShapeDtypeStruct((B, Oout, N), jnp.bfloat16),
                       jax.ShapeDtypeStruct((Oout, 2), f32)],
        )(x, st, g.reshape(Oin, 1), b.reshape(Oin, 1), W.astype(jnp.bfloat16))

    y1, st1 = bn_layer(y0, st0, gamma0, beta0, W1, O0, O1)
    y2, st2 = bn_layer(y1, st1, gamma1, beta1, W2, O1, O2)

    out = pl.pallas_call(
        functools.partial(_bn_relu_kernel, count=count),
        grid=grid,
        in_specs=[cm_spec(O2), pl.BlockSpec((O2, 2), lambda b, n: (0, 0)),
                  pl.BlockSpec((O2, 1), lambda b, n: (0, 0)),
                  pl.BlockSpec((O2, 1), lambda b, n: (0, 0))],
        out_specs=cm_spec(O2),
        out_shape=jax.ShapeDtypeStruct((B, O2, N), f32),
    )(y2, st2, gamma2.reshape(O2, 1), beta2.reshape(O2, 1))
    return out
